# baseline (speedup 1.0000x reference)
import functools

import jax
import jax.numpy as jnp
from jax import lax
from jax.experimental import pallas as pl
from jax.experimental.pallas import tpu as pltpu
from jax.experimental.pallas import tpu_sc as plsc

N = 10000
D = 256
HD = 128
E = 160000
CH = 128
NT = 16
NC = 2
CPT = 80
EPAD = NT * CPT * CH
NROW_PAD = N + 8
RPT = 624
RPT_LAST = N - RPT * (NT - 1)
NG = 16
LAYERS = 3


def _prep_edges(ei):
    src = ei[0]
    dst = ei[1]
    pad_n = EPAD - E
    ar = jnp.arange(pad_n, dtype=jnp.int32)
    dst_p = jnp.concatenate([dst, (ar * 997) % N])
    src_p = jnp.concatenate([src, N + (ar % 8)])
    return dst_p, src_p


def _sc_chain(x, dst3, src3):
    mesh = plsc.VectorSubcoreMesh(core_axis_name="c", subcore_axis_name="s")

    @functools.partial(
        pl.kernel,
        out_type=(jax.ShapeDtypeStruct((N, D), jnp.float32),
                  jax.ShapeDtypeStruct((N, D), jnp.float32),
                  jax.ShapeDtypeStruct((N, D), jnp.float32)),
        mesh=mesh,
        scratch_types=[
            pltpu.VMEM((CPT * CH,), jnp.int32),
            pltpu.VMEM((CH,), jnp.int32),
            pltpu.VMEM((CH,), jnp.int32),
            pltpu.VMEM_SHARED((NROW_PAD, HD), jnp.float32),
            pltpu.VMEM((CH, HD), jnp.float32),
            pltpu.VMEM((CH, HD), jnp.float32),
            pltpu.SemaphoreType.DMA,
            pltpu.SemaphoreType.DMA,
            pltpu.SemaphoreType.DMA,
            pltpu.SemaphoreType.DMA,
            pltpu.SemaphoreType.DMA,
            pltpu.SemaphoreType.DMA,
        ],
    )
    def chain(x_hbm, dst_hbm, src_hbm, out_hbm, t1_hbm, t2_hbm, idxg, srcb0,
              srcb1, acc, rows0, rows1, gsem0, gsem1, ssem0, ssem1, wsem0,
              wsem1):
        c = lax.axis_index("c")
        s = lax.axis_index("s")
        col = c * HD
        ebase = s * CPT * CH

        pltpu.sync_copy(dst_hbm.at[pl.ds(ebase, CPT * CH)], idxg)

        def fetch_src(j, buf, sem):
            pltpu.async_copy(src_hbm.at[pl.ds(ebase + j * CH, CH)], buf, sem)

        def fetch_src_wait(j, buf, sem):
            pltpu.make_async_copy(
                src_hbm.at[pl.ds(ebase + j * CH, CH)], buf, sem).wait()

        @pl.when(s < NT - 1)
        def _():
            pltpu.sync_copy(x_hbm.at[pl.ds(s * RPT, RPT), pl.ds(col, HD)],
                            acc.at[pl.ds(s * RPT, RPT)])

        @pl.when(s == NT - 1)
        def _():
            pltpu.sync_copy(
                x_hbm.at[pl.ds((NT - 1) * RPT, RPT_LAST), pl.ds(col, HD)],
                acc.at[pl.ds((NT - 1) * RPT, RPT_LAST)])

        def run_layer(xin_hbm, xout_hbm):
            def gather(j, buf, sem):
                return pltpu.async_copy(
                    xin_hbm.at[idxg.at[pl.ds(j * CH, CH)], pl.ds(col, HD)],
                    buf, sem)

            def gather_wait(j, buf, sem):
                pltpu.make_async_copy(
                    xin_hbm.at[idxg.at[pl.ds(j * CH, CH)], pl.ds(col, HD)],
                    buf, sem).wait()

            def scatter_wait(buf, ibuf, sem):
                pltpu.make_async_copy(buf, acc.at[ibuf], sem).wait()

            fetch_src(0, srcb0, ssem0)
            fetch_src(1, srcb1, ssem1)
            gather(0, rows0, gsem0)
            gather(1, rows1, gsem1)
            plsc.subcore_barrier()

            @pl.loop(0, CPT, step=2)
            def _(j):
                fetch_src_wait(j, srcb0, ssem0)
                gather_wait(j, rows0, gsem0)
                pltpu.async_copy(rows0, acc.at[srcb0], wsem0, add=True)

                fetch_src_wait(j + 1, srcb1, ssem1)
                gather_wait(j + 1, rows1, gsem1)
                pltpu.async_copy(rows1, acc.at[srcb1], wsem1, add=True)

                scatter_wait(rows0, srcb0, wsem0)

                @pl.when(j + 2 < CPT)
                def _():
                    fetch_src(j + 2, srcb0, ssem0)
                    gather(j + 2, rows0, gsem0)

                scatter_wait(rows1, srcb1, wsem1)

                @pl.when(j + 3 < CPT)
                def _():
                    fetch_src(j + 3, srcb1, ssem1)
                    gather(j + 3, rows1, gsem1)

            plsc.subcore_barrier()

            @pl.when(s < NT - 1)
            def _():
                pltpu.sync_copy(
                    acc.at[pl.ds(s * RPT, RPT)],
                    xout_hbm.at[pl.ds(s * RPT, RPT), pl.ds(col, HD)])

            @pl.when(s == NT - 1)
            def _():
                pltpu.sync_copy(
                    acc.at[pl.ds((NT - 1) * RPT, RPT_LAST)],
                    xout_hbm.at[pl.ds((NT - 1) * RPT, RPT_LAST),
                                pl.ds(col, HD)])

            plsc.subcore_barrier()

        run_layer(x_hbm, t1_hbm)
        run_layer(t1_hbm, t2_hbm)
        run_layer(t2_hbm, out_hbm)

    return chain(x, dst3, src3)[0]


BR = 400


def _merge(fw, bw, w1t, w2t, b, batch):
    def body(fw_ref, bw_ref, w1_ref, w2_ref, b_ref, bt_ref, hout_ref, gh_ref):
        i = pl.program_id(0)
        hb = jnp.dot(fw_ref[...], w1_ref[...],
                     preferred_element_type=jnp.float32)
        hb += jnp.dot(bw_ref[...], w2_ref[...],
                      preferred_element_type=jnp.float32)
        hb += b_ref[...][None, :]
        hout_ref[...] = hb

        @pl.when(i == 0)
        def _():
            gh_ref[...] = jnp.full((NG, D), -jnp.inf, jnp.float32)

        bt = bt_ref[...]
        neg = jnp.full((BR, D), -jnp.inf, jnp.float32)
        parts = []
        for g in range(NG):
            vals = jnp.where(bt == g, hb, neg)
            parts.append(jnp.max(vals, axis=0, keepdims=True))
        gh_ref[...] = jnp.maximum(gh_ref[...], jnp.concatenate(parts, axis=0))

    return pl.pallas_call(
        body,
        grid=(N // BR,),
        in_specs=[
            pl.BlockSpec((BR, D), lambda i: (i, 0)),
            pl.BlockSpec((BR, D), lambda i: (i, 0)),
            pl.BlockSpec((D, D), lambda i: (0, 0)),
            pl.BlockSpec((D, D), lambda i: (0, 0)),
            pl.BlockSpec((D,), lambda i: (0,)),
            pl.BlockSpec((BR, 1), lambda i: (i, 0)),
        ],
        out_specs=[
            pl.BlockSpec((BR, D), lambda i: (i, 0)),
            pl.BlockSpec((NG, D), lambda i: (0, 0)),
        ],
        out_shape=[
            jax.ShapeDtypeStruct((N, D), jnp.float32),
            jax.ShapeDtypeStruct((NG, D), jnp.float32),
        ],
    )(fw, bw, w1t, w2t, b, batch.reshape(N, 1))


def kernel(h, fw_edge_index, bw_edge_index, batch, W_merge, b_merge):
    dstf, srcf = _prep_edges(fw_edge_index)
    dstb, srcb = _prep_edges(bw_edge_index)
    x = _sc_chain(h, dstf, srcf)
    y = _sc_chain(h, dstb, srcb)
    w1t = W_merge[:, :D].T
    w2t = W_merge[:, D:].T
    h_out, g_h = _merge(x, y, w1t, w2t, b_merge, batch)
    return (g_h, h_out)


# speedup vs baseline: 1.1559x; 1.1559x over previous
import functools

import jax
import jax.numpy as jnp
from jax import lax
from jax.experimental import pallas as pl
from jax.experimental.pallas import tpu as pltpu
from jax.experimental.pallas import tpu_sc as plsc

N = 10000
D = 256
HD = 128
E = 160000
CH = 128
NT = 16
NC = 2
CPT = 80
EPAD = NT * CPT * CH
NROW_PAD = N + 8
RPT = 624
RPT_LAST = N - RPT * (NT - 1)
NG = 16
LAYERS = 3


def _prep_edges(ei):
    src = ei[0]
    dst = ei[1]
    pad_n = EPAD - E
    ar = jnp.arange(pad_n, dtype=jnp.int32)
    dst_p = jnp.concatenate([dst, (ar * 997) % N])
    src_p = jnp.concatenate([src, N + (ar % 8)])
    return dst_p, src_p


def _sc_chain(x, dst3, src3):
    mesh = plsc.VectorSubcoreMesh(core_axis_name="c", subcore_axis_name="s")

    @functools.partial(
        pl.kernel,
        out_type=(jax.ShapeDtypeStruct((N, D), jnp.float32),
                  jax.ShapeDtypeStruct((N, D), jnp.float32),
                  jax.ShapeDtypeStruct((N, D), jnp.float32)),
        mesh=mesh,
        scratch_types=[
            pltpu.VMEM((CPT * CH,), jnp.int32),
            pltpu.VMEM((CH,), jnp.int32),
            pltpu.VMEM((CH,), jnp.int32),
            pltpu.VMEM_SHARED((NROW_PAD, HD), jnp.float32),
            pltpu.VMEM((CH, HD), jnp.float32),
            pltpu.VMEM((CH, HD), jnp.float32),
            pltpu.SemaphoreType.DMA,
            pltpu.SemaphoreType.DMA,
            pltpu.SemaphoreType.DMA,
            pltpu.SemaphoreType.DMA,
            pltpu.SemaphoreType.DMA,
            pltpu.SemaphoreType.DMA,
        ],
    )
    def chain(x_hbm, dst_hbm, src_hbm, out_hbm, t1_hbm, t2_hbm, idxg, srcb0,
              srcb1, acc, rows0, rows1, gsem0, gsem1, ssem0, ssem1, wsem0,
              wsem1):
        c = lax.axis_index("c")
        s = lax.axis_index("s")
        col = c * HD
        ebase = s * CPT * CH

        pltpu.sync_copy(dst_hbm.at[pl.ds(ebase, CPT * CH)], idxg)

        def fetch_src(j, buf, sem):
            pltpu.async_copy(src_hbm.at[pl.ds(ebase + j * CH, CH)], buf, sem)

        def fetch_src_wait(j, buf, sem):
            pltpu.make_async_copy(
                src_hbm.at[pl.ds(ebase + j * CH, CH)], buf, sem).wait()

        @pl.when(s < NT - 1)
        def _():
            pltpu.sync_copy(x_hbm.at[pl.ds(s * RPT, RPT), pl.ds(col, HD)],
                            acc.at[pl.ds(s * RPT, RPT)])

        @pl.when(s == NT - 1)
        def _():
            pltpu.sync_copy(
                x_hbm.at[pl.ds((NT - 1) * RPT, RPT_LAST), pl.ds(col, HD)],
                acc.at[pl.ds((NT - 1) * RPT, RPT_LAST)])

        def run_layer(xin_hbm, xout_hbm):
            def gather(j, buf, sem):
                return pltpu.async_copy(
                    xin_hbm.at[idxg.at[pl.ds(j * CH, CH)], pl.ds(col, HD)],
                    buf, sem)

            def gather_wait(j, buf, sem):
                pltpu.make_async_copy(
                    xin_hbm.at[idxg.at[pl.ds(j * CH, CH)], pl.ds(col, HD)],
                    buf, sem).wait()

            def scatter_wait(buf, ibuf, sem):
                pltpu.make_async_copy(buf, acc.at[ibuf], sem).wait()

            fetch_src(0, srcb0, ssem0)
            fetch_src(1, srcb1, ssem1)
            gather(0, rows0, gsem0)
            gather(1, rows1, gsem1)
            plsc.subcore_barrier()

            @pl.loop(0, CPT, step=2)
            def _(j):
                fetch_src_wait(j, srcb0, ssem0)
                gather_wait(j, rows0, gsem0)
                pltpu.sync_copy(rows0, acc.at[srcb0], add=True)

                @pl.when(j + 2 < CPT)
                def _():
                    fetch_src(j + 2, srcb0, ssem0)
                    gather(j + 2, rows0, gsem0)

                fetch_src_wait(j + 1, srcb1, ssem1)
                gather_wait(j + 1, rows1, gsem1)
                pltpu.sync_copy(rows1, acc.at[srcb1], add=True)

                @pl.when(j + 3 < CPT)
                def _():
                    fetch_src(j + 3, srcb1, ssem1)
                    gather(j + 3, rows1, gsem1)

            plsc.subcore_barrier()

            @pl.when(s < NT - 1)
            def _():
                pltpu.sync_copy(
                    acc.at[pl.ds(s * RPT, RPT)],
                    xout_hbm.at[pl.ds(s * RPT, RPT), pl.ds(col, HD)])

            @pl.when(s == NT - 1)
            def _():
                pltpu.sync_copy(
                    acc.at[pl.ds((NT - 1) * RPT, RPT_LAST)],
                    xout_hbm.at[pl.ds((NT - 1) * RPT, RPT_LAST),
                                pl.ds(col, HD)])

            plsc.subcore_barrier()

        run_layer(x_hbm, t1_hbm)
        run_layer(t1_hbm, t2_hbm)
        run_layer(t2_hbm, out_hbm)

    return chain(x, dst3, src3)[0]


BR = 400


def _merge(fw, bw, w1t, w2t, b, batch):
    def body(fw_ref, bw_ref, w1_ref, w2_ref, b_ref, bt_ref, hout_ref, gh_ref):
        i = pl.program_id(0)
        hb = jnp.dot(fw_ref[...], w1_ref[...],
                     preferred_element_type=jnp.float32)
        hb += jnp.dot(bw_ref[...], w2_ref[...],
                      preferred_element_type=jnp.float32)
        hb += b_ref[...][None, :]
        hout_ref[...] = hb

        @pl.when(i == 0)
        def _():
            gh_ref[...] = jnp.full((NG, D), -jnp.inf, jnp.float32)

        bt = bt_ref[...]
        neg = jnp.full((BR, D), -jnp.inf, jnp.float32)
        parts = []
        for g in range(NG):
            vals = jnp.where(bt == g, hb, neg)
            parts.append(jnp.max(vals, axis=0, keepdims=True))
        gh_ref[...] = jnp.maximum(gh_ref[...], jnp.concatenate(parts, axis=0))

    return pl.pallas_call(
        body,
        grid=(N // BR,),
        in_specs=[
            pl.BlockSpec((BR, D), lambda i: (i, 0)),
            pl.BlockSpec((BR, D), lambda i: (i, 0)),
            pl.BlockSpec((D, D), lambda i: (0, 0)),
            pl.BlockSpec((D, D), lambda i: (0, 0)),
            pl.BlockSpec((D,), lambda i: (0,)),
            pl.BlockSpec((BR, 1), lambda i: (i, 0)),
        ],
        out_specs=[
            pl.BlockSpec((BR, D), lambda i: (i, 0)),
            pl.BlockSpec((NG, D), lambda i: (0, 0)),
        ],
        out_shape=[
            jax.ShapeDtypeStruct((N, D), jnp.float32),
            jax.ShapeDtypeStruct((NG, D), jnp.float32),
        ],
    )(fw, bw, w1t, w2t, b, batch.reshape(N, 1))


def kernel(h, fw_edge_index, bw_edge_index, batch, W_merge, b_merge):
    dstf, srcf = _prep_edges(fw_edge_index)
    dstb, srcb = _prep_edges(bw_edge_index)
    x = _sc_chain(h, dstf, srcf)
    y = _sc_chain(h, dstb, srcb)
    w1t = W_merge[:, :D].T
    w2t = W_merge[:, D:].T
    h_out, g_h = _merge(x, y, w1t, w2t, b_merge, batch)
    return (g_h, h_out)
